# baseline (speedup 1.0000x reference)
import functools

import jax
import jax.numpy as jnp
from jax import lax
from jax.experimental import pallas as pl
from jax.experimental.pallas import tpu as pltpu
from jax.experimental.pallas import tpu_sc as plsc

_STRIDE = 1000
_D = 64
_B = 16384
_NC = 2
_NS = 16
_NW = _NC * _NS
_BPW = _B // _NW
_L = 16
_CHUNK = 128
_NCHUNK = _BPW // _CHUNK


def _make_gather():
    mesh = plsc.VectorSubcoreMesh(core_axis_name="c", subcore_axis_name="s")

    @functools.partial(
        pl.kernel,
        mesh=mesh,
        out_type=jax.ShapeDtypeStruct((_B, _D), jnp.float32),
        compiler_params=pltpu.CompilerParams(use_tc_tiling_on_sc=False),
        scratch_types=[
            pltpu.VMEM((_BPW,), jnp.int32),
            pltpu.VMEM((_BPW,), jnp.int32),
            pltpu.VMEM((_NCHUNK, _CHUNK), jnp.int32),
            pltpu.VMEM((_BPW, _D), jnp.float32),
            pltpu.SemaphoreType.DMA,
        ],
    )
    def gather(x1_hbm, x2_hbm, w_hbm, out_hbm, x1_v, x2_v, idx_v, rows_v, sem):
        wid = lax.axis_index("s") * _NC + lax.axis_index("c")
        base = wid * _BPW
        pltpu.sync_copy(x1_hbm.at[pl.ds(base, _BPW)], x1_v)
        pltpu.sync_copy(x2_hbm.at[pl.ds(base, _BPW)], x2_v)
        for i in range(_BPW // _L):
            a = x1_v[pl.ds(i * _L, _L)]
            b = x2_v[pl.ds(i * _L, _L)]
            row = (i * _L) // _CHUNK
            col = (i * _L) % _CHUNK
            idx_v[row, pl.ds(col, _L)] = a * _STRIDE + b
        copies = [
            pltpu.async_copy(
                w_hbm.at[idx_v.at[j]],
                rows_v.at[pl.ds(j * _CHUNK, _CHUNK)],
                sem,
            )
            for j in range(_NCHUNK)
        ]
        for cp in copies:
            cp.wait()
        pltpu.sync_copy(rows_v, out_hbm.at[pl.ds(base, _BPW)])

    return gather


_gather = _make_gather()


@jax.jit
def kernel(x1, x2, weights):
    out = _gather(x1.astype(jnp.int32), x2.astype(jnp.int32), weights)
    return out.reshape(_B, _D)


# speedup vs baseline: 1.7051x; 1.7051x over previous
import functools

import jax
import jax.numpy as jnp
from jax import lax
from jax.experimental import pallas as pl
from jax.experimental.pallas import tpu as pltpu
from jax.experimental.pallas import tpu_sc as plsc

_STRIDE = 1000
_D = 64
_B = 16384
_NC = 2
_NS = 16
_NW = _NC * _NS
_BPW = _B // _NW
_L = 16


def _make_gather():
    mesh = plsc.VectorSubcoreMesh(core_axis_name="c", subcore_axis_name="s")

    @functools.partial(
        pl.kernel,
        mesh=mesh,
        out_type=jax.ShapeDtypeStruct((_B, _D), jnp.float32),
        scratch_types=[
            pltpu.VMEM((_BPW,), jnp.int32),
            pltpu.VMEM((_BPW,), jnp.int32),
            pltpu.VMEM((_BPW, _D), jnp.float32),
            pltpu.SemaphoreType.DMA,
        ],
    )
    def gather(x1_hbm, x2_hbm, w_hbm, out_hbm, x1_v, x2_v, rows_v, sem):
        wid = lax.axis_index("s") * _NC + lax.axis_index("c")
        base = wid * _BPW
        pltpu.sync_copy(x1_hbm.at[pl.ds(base, _BPW)], x1_v)
        pltpu.sync_copy(x2_hbm.at[pl.ds(base, _BPW)], x2_v)
        copies = []
        for m in range(_BPW // _L):
            a = x1_v[pl.ds(m * _L, _L)]
            b = x2_v[pl.ds(m * _L, _L)]
            fused = a * _STRIDE + b
            for l in range(_L):
                s = lax.squeeze(lax.slice(fused, (l,), (l + 1,)), (0,))
                e = m * _L + l
                copies.append(
                    pltpu.async_copy(w_hbm.at[s], rows_v.at[e], sem)
                )
        for cp in copies:
            cp.wait()
        pltpu.sync_copy(rows_v, out_hbm.at[pl.ds(base, _BPW)])

    return gather


_gather = _make_gather()


@jax.jit
def kernel(x1, x2, weights):
    out = _gather(x1.astype(jnp.int32), x2.astype(jnp.int32), weights)
    return out.reshape(_B, _D)
